# baseline (speedup 1.0000x reference)
import functools

import jax
import jax.numpy as jnp
from jax import lax
from jax.experimental import pallas as pl
from jax.experimental.pallas import tpu as pltpu
from jax.experimental.pallas import tpu_sc as plsc

N_NODES = 10000
K = 32
D = 128
HID = 32
OUT_U = 16
H = 4
E = N_NODES * K

_BN1 = 2000


def _mm_body(x_ref, w_ref, o_ref):
    o_ref[:] = jnp.dot(
        x_ref[:], w_ref[:], preferred_element_type=jnp.float32
    ).astype(jnp.bfloat16)


def _compute_pn(x, w1n):
    return pl.pallas_call(
        _mm_body,
        grid=(N_NODES // _BN1,),
        in_specs=[
            pl.BlockSpec((_BN1, D), lambda i: (i, 0)),
            pl.BlockSpec((D, HID), lambda i: (0, 0)),
        ],
        out_specs=pl.BlockSpec((_BN1, HID), lambda i: (i, 0)),
        out_shape=jax.ShapeDtypeStruct((N_NODES, HID), jnp.bfloat16),
    )(x, w1n)


_CH = 1000


def _gather_body(nc, idx_hbm, pn_hbm, out_hbm, idx_all, rows0, rows1, sg0, sg1, ss0, ss1):
    c = lax.axis_index("c")
    s = lax.axis_index("s")
    wid = s * nc + c
    e_per_w = E // (nc * 16)
    nch = e_per_w // _CH
    base = wid * e_per_w
    pltpu.sync_copy(idx_hbm.at[pl.ds(base, e_per_w)], idx_all)
    rows = (rows0, rows1)
    gsem = (sg0, sg1)
    ssem = (ss0, ss1)
    gd = [None, None]
    sd = [None, None]
    for i in range(nch):
        b = i & 1
        if sd[b] is not None:
            sd[b].wait()
        gd[b] = pltpu.async_copy(
            pn_hbm.at[idx_all.at[pl.ds(i * _CH, _CH)]], rows[b], gsem[b]
        )
        if i >= 1:
            pb = 1 - b
            gd[pb].wait()
            sd[pb] = pltpu.async_copy(
                rows[pb], out_hbm.at[pl.ds(base + (i - 1) * _CH, _CH)], ssem[pb]
            )
    lb = (nch - 1) & 1
    gd[lb].wait()
    sd[lb] = pltpu.async_copy(
        rows[lb], out_hbm.at[pl.ds(base + (nch - 1) * _CH, _CH)], ssem[lb]
    )
    sd[0].wait()
    sd[1].wait()


def _gather(idx_flat, pn):
    info = plsc.get_sparse_core_info()
    e_per_w = E // (info.num_cores * info.num_subcores)
    mesh = plsc.VectorSubcoreMesh(core_axis_name="c", subcore_axis_name="s")
    fn = pl.kernel(
        functools.partial(_gather_body, info.num_cores),
        mesh=mesh,
        out_type=jax.ShapeDtypeStruct((E, HID), jnp.bfloat16),
        scratch_types=[
            pltpu.VMEM((e_per_w,), jnp.int32),
            pltpu.VMEM((_CH, HID), jnp.bfloat16),
            pltpu.VMEM((_CH, HID), jnp.bfloat16),
            pltpu.SemaphoreType.DMA,
            pltpu.SemaphoreType.DMA,
            pltpu.SemaphoreType.DMA,
            pltpu.SemaphoreType.DMA,
        ],
        compiler_params=pltpu.CompilerParams(use_tc_tiling_on_sc=False, skip_device_barrier=True),
    )
    return fn(idx_flat, pn)


_BN3 = 400


def _agg_body(
    g_ref, x_ref, w1c_ref, t32_ref, b1_ref, w2bd_ref, b2_ref, wabd_ref, ba_ref,
    eexp_ref, sden_ref, masm_ref, o_ref,
):
    f32 = jnp.float32
    pc = jnp.dot(x_ref[:], w1c_ref[:], preferred_element_type=f32)
    pc_t = jnp.dot(pc, t32_ref[:], preferred_element_type=f32)
    h = jnp.maximum(g_ref[:].astype(jnp.float32) + pc_t + b1_ref[:], 0.0)
    t = jnp.maximum(
        jnp.dot(h, w2bd_ref[:], preferred_element_type=f32) + b2_ref[:], 0.0
    )
    att = jnp.maximum(
        jnp.dot(t, wabd_ref[:], preferred_element_type=f32) + ba_ref[:], 0.0
    )
    ex = jnp.exp(att)
    den = jnp.dot(ex, sden_ref[:], preferred_element_type=f32)
    pp = ex / den
    outs = []
    for a in range(H):
        w = jnp.dot(
            pp[:, K * a : K * (a + 1)], eexp_ref[:], preferred_element_type=f32
        )
        outs.append(
            jnp.dot(w * t, masm_ref[:], preferred_element_type=f32)
        )
    o_ref[:] = jnp.concatenate(outs, axis=1)


def _aggregate(gp, x, w1c, t32, b1p, w2bd, b2p, wabd, bap, eexp, sden, masm,
               interpret=False):
    nb = N_NODES // _BN3
    full = lambda shape: pl.BlockSpec(shape, lambda i: tuple(0 for _ in shape))
    return pl.pallas_call(
        _agg_body,
        grid=(nb,),
        in_specs=[
            pl.BlockSpec((_BN3, K * HID), lambda i: (i, 0)),
            pl.BlockSpec((_BN3, D), lambda i: (i, 0)),
            full((D, HID)),
            full((HID, K * HID)),
            full((1, K * HID)),
            full((K * HID, K * OUT_U)),
            full((1, K * OUT_U)),
            full((K * OUT_U, K * H)),
            full((1, K * H)),
            full((K, K * OUT_U)),
            full((K * H, K * H)),
            full((K * OUT_U, OUT_U)),
        ],
        out_specs=pl.BlockSpec((_BN3, H * OUT_U), lambda i: (i, 0)),
        out_shape=jax.ShapeDtypeStruct((N_NODES, H * OUT_U), jnp.float32),
        interpret=interpret,
    )(gp, x, w1c, t32, b1p, w2bd, b2p, wabd, bap, eexp, sden, masm)


def _stage3_constants(W1, b1, W2, b2, Wa, ba):
    f32 = jnp.float32
    w1c = W1[D:]
    t32 = jnp.kron(jnp.ones((1, K), f32), jnp.eye(HID, dtype=f32))
    b1p = jnp.tile(b1, K).reshape(1, K * HID)
    w2bd = jnp.kron(jnp.eye(K, dtype=f32), W2)
    b2p = jnp.tile(b2, K).reshape(1, K * OUT_U)
    wabd = jnp.kron(jnp.eye(K, dtype=f32), Wa)
    bap = jnp.tile(ba, K).reshape(1, K * H)
    eexp = jnp.kron(jnp.eye(K, dtype=f32), jnp.ones((1, OUT_U), f32))
    sden = jnp.kron(jnp.eye(K, dtype=f32), jnp.ones((H, H), f32))
    masm = jnp.kron(jnp.ones((K, 1), f32), jnp.eye(OUT_U, dtype=f32))
    return w1c, t32, b1p, w2bd, b2p, wabd, bap, eexp, sden, masm


def kernel(x, neighbor_idx, W1, b1, W2, b2, Wa, ba):
    w1n = W1[:D]
    pn = _compute_pn(x, w1n)
    idx_flat = neighbor_idx.reshape(-1).astype(jnp.int32)
    g = _gather(idx_flat, pn)
    gp = g.reshape(N_NODES, K * HID)
    consts = _stage3_constants(W1, b1, W2, b2, Wa, ba)
    return _aggregate(gp, x, *consts)


# speedup vs baseline: 1.2129x; 1.2129x over previous
import functools

import jax
import jax.numpy as jnp
from jax import lax
from jax.experimental import pallas as pl
from jax.experimental.pallas import tpu as pltpu
from jax.experimental.pallas import tpu_sc as plsc

N_NODES = 10000
K = 32
D = 128
HID = 32
OUT_U = 16
H = 4
E = N_NODES * K

_BN1 = 2000


def _mm_body(x_ref, w_ref, o_ref):
    o_ref[:] = jnp.dot(x_ref[:], w_ref[:], preferred_element_type=jnp.float32)


def _compute_pn(x, w1n):
    return pl.pallas_call(
        _mm_body,
        grid=(N_NODES // _BN1,),
        in_specs=[
            pl.BlockSpec((_BN1, D), lambda i: (i, 0)),
            pl.BlockSpec((D, HID), lambda i: (0, 0)),
        ],
        out_specs=pl.BlockSpec((_BN1, HID), lambda i: (i, 0)),
        out_shape=jax.ShapeDtypeStruct((N_NODES, HID), jnp.float32),
    )(x, w1n)


_CH = 1000


def _gather_body(nc, idx_hbm, pn_hbm, out_hbm, idx_all, rows0, rows1, sg0, sg1, ss0, ss1):
    c = lax.axis_index("c")
    s = lax.axis_index("s")
    wid = s * nc + c
    e_per_w = E // (nc * 16)
    nch = e_per_w // _CH
    base = wid * e_per_w
    pltpu.sync_copy(idx_hbm.at[pl.ds(base, e_per_w)], idx_all)
    rows = (rows0, rows1)
    gsem = (sg0, sg1)
    ssem = (ss0, ss1)
    gd = [None, None]
    sd = [None, None]
    for i in range(nch):
        b = i & 1
        if sd[b] is not None:
            sd[b].wait()
        gd[b] = pltpu.async_copy(
            pn_hbm.at[idx_all.at[pl.ds(i * _CH, _CH)]], rows[b], gsem[b]
        )
        if i >= 1:
            pb = 1 - b
            gd[pb].wait()
            sd[pb] = pltpu.async_copy(
                rows[pb], out_hbm.at[pl.ds(base + (i - 1) * _CH, _CH)], ssem[pb]
            )
    lb = (nch - 1) & 1
    gd[lb].wait()
    sd[lb] = pltpu.async_copy(
        rows[lb], out_hbm.at[pl.ds(base + (nch - 1) * _CH, _CH)], ssem[lb]
    )
    sd[0].wait()
    sd[1].wait()


def _gather(idx_flat, pn):
    info = plsc.get_sparse_core_info()
    e_per_w = E // (info.num_cores * info.num_subcores)
    mesh = plsc.VectorSubcoreMesh(core_axis_name="c", subcore_axis_name="s")
    fn = pl.kernel(
        functools.partial(_gather_body, info.num_cores),
        mesh=mesh,
        out_type=jax.ShapeDtypeStruct((E, HID), jnp.float32),
        scratch_types=[
            pltpu.VMEM((e_per_w,), jnp.int32),
            pltpu.VMEM((_CH, HID), jnp.float32),
            pltpu.VMEM((_CH, HID), jnp.float32),
            pltpu.SemaphoreType.DMA,
            pltpu.SemaphoreType.DMA,
            pltpu.SemaphoreType.DMA,
            pltpu.SemaphoreType.DMA,
        ],
        compiler_params=pltpu.CompilerParams(use_tc_tiling_on_sc=False, skip_device_barrier=True),
    )
    return fn(idx_flat, pn)


_BN3 = 400


def _agg_body(
    g_ref, x_ref, w1c_ref, t32_ref, b1_ref, w2bd_ref, b2_ref, wabd_ref, ba_ref,
    eexp_ref, sden_ref, masm_ref, o_ref,
):
    f32 = jnp.float32
    pc = jnp.dot(x_ref[:], w1c_ref[:], preferred_element_type=f32)
    pc_t = jnp.dot(pc, t32_ref[:], preferred_element_type=f32)
    h = jnp.maximum(g_ref[:] + pc_t + b1_ref[:], 0.0)
    t = jnp.maximum(
        jnp.dot(h, w2bd_ref[:], preferred_element_type=f32) + b2_ref[:], 0.0
    )
    att = jnp.maximum(
        jnp.dot(t, wabd_ref[:], preferred_element_type=f32) + ba_ref[:], 0.0
    )
    ex = jnp.exp(att)
    den = jnp.dot(ex, sden_ref[:], preferred_element_type=f32)
    pp = ex / den
    outs = []
    for a in range(H):
        w = jnp.dot(
            pp[:, K * a : K * (a + 1)], eexp_ref[:], preferred_element_type=f32
        )
        outs.append(
            jnp.dot(w * t, masm_ref[:], preferred_element_type=f32)
        )
    o_ref[:] = jnp.concatenate(outs, axis=1)


def _aggregate(gp, x, w1c, t32, b1p, w2bd, b2p, wabd, bap, eexp, sden, masm,
               interpret=False):
    nb = N_NODES // _BN3
    full = lambda shape: pl.BlockSpec(shape, lambda i: tuple(0 for _ in shape))
    return pl.pallas_call(
        _agg_body,
        grid=(nb,),
        in_specs=[
            pl.BlockSpec((_BN3, K * HID), lambda i: (i, 0)),
            pl.BlockSpec((_BN3, D), lambda i: (i, 0)),
            full((D, HID)),
            full((HID, K * HID)),
            full((1, K * HID)),
            full((K * HID, K * OUT_U)),
            full((1, K * OUT_U)),
            full((K * OUT_U, K * H)),
            full((1, K * H)),
            full((K, K * OUT_U)),
            full((K * H, K * H)),
            full((K * OUT_U, OUT_U)),
        ],
        out_specs=pl.BlockSpec((_BN3, H * OUT_U), lambda i: (i, 0)),
        out_shape=jax.ShapeDtypeStruct((N_NODES, H * OUT_U), jnp.float32),
        interpret=interpret,
    )(gp, x, w1c, t32, b1p, w2bd, b2p, wabd, bap, eexp, sden, masm)


def _stage3_constants(W1, b1, W2, b2, Wa, ba):
    f32 = jnp.float32
    w1c = W1[D:]
    t32 = jnp.kron(jnp.ones((1, K), f32), jnp.eye(HID, dtype=f32))
    b1p = jnp.tile(b1, K).reshape(1, K * HID)
    w2bd = jnp.kron(jnp.eye(K, dtype=f32), W2)
    b2p = jnp.tile(b2, K).reshape(1, K * OUT_U)
    wabd = jnp.kron(jnp.eye(K, dtype=f32), Wa)
    bap = jnp.tile(ba, K).reshape(1, K * H)
    eexp = jnp.kron(jnp.eye(K, dtype=f32), jnp.ones((1, OUT_U), f32))
    sden = jnp.kron(jnp.eye(K, dtype=f32), jnp.ones((H, H), f32))
    masm = jnp.kron(jnp.ones((K, 1), f32), jnp.eye(OUT_U, dtype=f32))
    return w1c, t32, b1p, w2bd, b2p, wabd, bap, eexp, sden, masm


def kernel(x, neighbor_idx, W1, b1, W2, b2, Wa, ba):
    w1n = W1[:D]
    pn = _compute_pn(x, w1n)
    idx_flat = neighbor_idx.reshape(-1).astype(jnp.int32)
    g = _gather(idx_flat, pn)
    gp = g.reshape(N_NODES, K * HID)
    consts = _stage3_constants(W1, b1, W2, b2, Wa, ba)
    return _aggregate(gp, x, *consts)
